# baseline (speedup 1.0000x reference)
import functools

import jax
import jax.numpy as jnp
from jax import lax
from jax.experimental import pallas as pl
from jax.experimental.pallas import tpu as pltpu
from jax.experimental.pallas import tpu_sc as plsc

N = 32768
NUM_MIX = 8
NC = 2
NS = 16
LANES = 16
NW = NC * NS
CHUNK = N // NW
NSLICE = CHUNK // LANES
U = 1
NGROUP = NSLICE // U

_A_OFF = 0
_W2_OFF = 8
_B2_OFF = _W2_OFF + 64
_W3_OFF = _B2_OFF + 8
_B3_OFF = _W3_OFF + 8
_WP_ROWS = _B3_OFF + 1


def _sc_body(obs_hbm, z_hbm, ctm_hbm, cts_hbm, wpm_hbm, wps_hbm,
             mean_hbm, std_hbm,
             obs_v, z_v, ctm_v, cts_v, wpm_v, wps_v, om_v, os_v):
    wid = lax.axis_index("c") * NS + lax.axis_index("s")
    base = wid * CHUNK

    pltpu.sync_copy(obs_hbm.at[pl.ds(base, CHUNK)], obs_v)
    pltpu.sync_copy(z_hbm.at[pl.ds(base, CHUNK)], z_v)
    pltpu.sync_copy(ctm_hbm, ctm_v)
    pltpu.sync_copy(cts_hbm, cts_v)
    pltpu.sync_copy(wpm_hbm, wpm_v)
    pltpu.sync_copy(wps_hbm, wps_v)

    def run_net(ct_v, wp_v, out_v, is_std):
        def row(r):
            return wp_v[pl.ds(r * LANES, LANES)]

        @plsc.parallel_loop(0, NGROUP, unroll=2)
        def group_body(g):
            o0 = g * (U * LANES)
            obs16 = []
            z8 = []
            for u in range(U):
                o = o0 + u * LANES
                obs16.append(obs_v[pl.ds(o, LANES)])
                z8.append(z_v[pl.ds(o, LANES)])

            s1 = [[None] * NUM_MIX for _ in range(U)]
            for j in range(NUM_MIX):
                aj = row(_A_OFF + j)
                for u in range(U):
                    cz = plsc.load_gather(ct_v, [z8[u] + j])
                    t = jnp.exp(obs16[u] * aj + cz)
                    s1[u][j] = 1.0 / (t + 1.0)

            out = [row(_B3_OFF)] * U
            for i in range(NUM_MIX):
                acc = [row(_B2_OFF + i)] * U
                for j in range(NUM_MIX):
                    w = row(_W2_OFF + i * NUM_MIX + j)
                    for u in range(U):
                        acc[u] = acc[u] + w * s1[u][j]
                w3 = row(_W3_OFF + i)
                for u in range(U):
                    t = jnp.exp(acc[u])
                    out[u] = out[u] + w3 * (1.0 / (t + 1.0))

            for u in range(U):
                o = o0 + u * LANES
                out_v[pl.ds(o, LANES)] = jnp.exp(out[u]) if is_std else out[u]

    run_net(ctm_v, wpm_v, om_v, False)
    run_net(cts_v, wps_v, os_v, True)

    pltpu.sync_copy(om_v, mean_hbm.at[pl.ds(base, CHUNK)])
    pltpu.sync_copy(os_v, std_hbm.at[pl.ds(base, CHUNK)])


def _scratch_types():
    return [
        pltpu.VMEM((CHUNK,), jnp.float32),
        pltpu.VMEM((CHUNK,), jnp.int32),
        pltpu.VMEM((NUM_MIX * NUM_MIX,), jnp.float32),
        pltpu.VMEM((NUM_MIX * NUM_MIX,), jnp.float32),
        pltpu.VMEM((_WP_ROWS * LANES,), jnp.float32),
        pltpu.VMEM((_WP_ROWS * LANES,), jnp.float32),
        pltpu.VMEM((CHUNK,), jnp.float32),
        pltpu.VMEM((CHUNK,), jnp.float32),
    ]


@functools.cache
def _sc_call():
    return functools.partial(
        pl.kernel,
        out_type=(
            jax.ShapeDtypeStruct((N,), jnp.float32),
            jax.ShapeDtypeStruct((N,), jnp.float32),
        ),
        mesh=plsc.VectorSubcoreMesh(
            core_axis_name="c", subcore_axis_name="s",
            num_cores=NC, num_subcores=NS,
        ),
        scratch_types=_scratch_types(),
        compiler_params=pltpu.CompilerParams(needs_layout_passes=False),
    )(_sc_body)


def _pack_net(W1, b1, W2, b2, W3, b3):
    a1 = -2.0 * W1[:, 0]
    ct = (-2.0 * (W1[:, 1:].T + b1[None, :])).reshape(-1)
    w2p = -4.0 * W2
    b2p = -2.0 * b2 + 2.0 * W2.sum(axis=1)
    w3p = 2.0 * W3[0]
    b3p = b3 - W3[0].sum()
    wp = jnp.concatenate([
        jnp.repeat(a1, LANES),
        jnp.repeat(w2p.reshape(-1), LANES),
        jnp.repeat(b2p, LANES),
        jnp.repeat(w3p, LANES),
        jnp.repeat(b3p, LANES),
    ])
    return ct, wp


def kernel(obs, k, z, mW1, mb1, mW2, mb2, mW3, mb3,
           sW1, sb1, sW2, sb2, sW3, sb3):
    del k
    ctm, wpm = _pack_net(mW1, mb1, mW2, mb2, mW3, mb3)
    cts, wps = _pack_net(sW1, sb1, sW2, sb2, sW3, sb3)
    mean, std = _sc_call()(
        obs, z.astype(jnp.int32) * NUM_MIX, ctm, cts, wpm, wps
    )
    return mean, std


# speedup vs baseline: 1.2032x; 1.2032x over previous
import functools

import jax
import jax.numpy as jnp
from jax import lax
from jax.experimental import pallas as pl
from jax.experimental.pallas import tpu as pltpu
from jax.experimental.pallas import tpu_sc as plsc

N = 32768
NUM_MIX = 8
NC = 2
NS = 16
LANES = 16
NW = NC * NS
CHUNK = N // NW
NSLICE = CHUNK // LANES

_CT_F = NUM_MIX * NUM_MIX
_A_OFF = 0
_W2_OFF = 8
_B2_OFF = _W2_OFF + 64
_W3_OFF = _B2_OFF + 8
_B3_OFF = _W3_OFF + 8
_WP_ROWS = _B3_OFF + 1
_NET_F = _CT_F + _WP_ROWS * LANES
_WTOT = 2 * _NET_F


def _sc_body(data_hbm, w_hbm, mean_hbm, std_hbm, data_v, w_v, out_v):
    wid = lax.axis_index("c") * NS + lax.axis_index("s")
    base = wid * CHUNK

    pltpu.sync_copy(data_hbm.at[pl.ds(base, CHUNK)], data_v.at[pl.ds(0, CHUNK)])
    pltpu.sync_copy(data_hbm.at[pl.ds(N + base, CHUNK)],
                    data_v.at[pl.ds(CHUNK, CHUNK)])
    pltpu.sync_copy(w_hbm, w_v)

    def run_net(nb, carry):
        ctb = nb * _NET_F
        wpb = ctb + _CT_F
        ob = nb * CHUNK

        def row(r):
            return w_v[pl.ds(wpb + r * LANES, LANES)]

        @plsc.parallel_loop(0, NSLICE, unroll=1)
        def slice_body(s):
            o = s * LANES
            obs16 = data_v[pl.ds(o, LANES)]
            z8 = plsc.bitcast(data_v[pl.ds(CHUNK + o, LANES)], jnp.int32) + ctb

            s1 = []
            for j in range(NUM_MIX):
                cz = plsc.load_gather(w_v, [z8 + j])
                t = jnp.exp(obs16 * row(_A_OFF + j) + cz)
                s1.append(1.0 / (t + 1.0))

            out = row(_B3_OFF)
            for i in range(NUM_MIX):
                acc = row(_B2_OFF + i)
                for j in range(NUM_MIX):
                    acc = acc + row(_W2_OFF + i * NUM_MIX + j) * s1[j]
                t = jnp.exp(acc)
                out = out + row(_W3_OFF + i) * (1.0 / (t + 1.0))

            out_v[pl.ds(ob + o, LANES)] = out

        return carry

    lax.fori_loop(0, 2, run_net, 0)

    @plsc.parallel_loop(0, NSLICE, unroll=1)
    def exp_body(s):
        o = CHUNK + s * LANES
        out_v[pl.ds(o, LANES)] = jnp.exp(out_v[pl.ds(o, LANES)])

    pltpu.sync_copy(out_v.at[pl.ds(0, CHUNK)], mean_hbm.at[pl.ds(base, CHUNK)])
    pltpu.sync_copy(out_v.at[pl.ds(CHUNK, CHUNK)],
                    std_hbm.at[pl.ds(base, CHUNK)])


def _scratch_types():
    return [
        pltpu.VMEM((2 * CHUNK,), jnp.float32),
        pltpu.VMEM((_WTOT,), jnp.float32),
        pltpu.VMEM((2 * CHUNK,), jnp.float32),
    ]


@functools.cache
def _sc_call():
    return functools.partial(
        pl.kernel,
        out_type=(
            jax.ShapeDtypeStruct((N,), jnp.float32),
            jax.ShapeDtypeStruct((N,), jnp.float32),
        ),
        mesh=plsc.VectorSubcoreMesh(
            core_axis_name="c", subcore_axis_name="s",
            num_cores=NC, num_subcores=NS,
        ),
        scratch_types=_scratch_types(),
        compiler_params=pltpu.CompilerParams(needs_layout_passes=False),
    )(_sc_body)


def _pack_net(W1, b1, W2, b2, W3, b3):
    a1 = -2.0 * W1[:, 0]
    ct = (-2.0 * (W1[:, 1:].T + b1[None, :])).reshape(-1)
    w2p = -4.0 * W2
    b2p = -2.0 * b2 + 2.0 * W2.sum(axis=1)
    w3p = 2.0 * W3[0]
    b3p = b3 - W3[0].sum()
    return jnp.concatenate([
        ct,
        jnp.repeat(a1, LANES),
        jnp.repeat(w2p.reshape(-1), LANES),
        jnp.repeat(b2p, LANES),
        jnp.repeat(w3p, LANES),
        jnp.repeat(b3p, LANES),
    ])


def kernel(obs, k, z, mW1, mb1, mW2, mb2, mW3, mb3,
           sW1, sb1, sW2, sb2, sW3, sb3):
    del k
    wbuf = jnp.concatenate([
        _pack_net(mW1, mb1, mW2, mb2, mW3, mb3),
        _pack_net(sW1, sb1, sW2, sb2, sW3, sb3),
    ])
    data = jnp.concatenate([
        obs,
        jax.lax.bitcast_convert_type(z.astype(jnp.int32) * NUM_MIX,
                                     jnp.float32),
    ])
    mean, std = _sc_call()(data, wbuf)
    return mean, std
